# Initial kernel scaffold; baseline (speedup 1.0000x reference)
import jax
import jax.numpy as jnp
from jax.experimental import pallas as pl


def kernel(x, edge_index, edge_attr, batch, Ws, Wd, We, att, gat_b, Wp, bp, smiles_repr, Wq, bq, Wk, bk, Wv, bv, Wo, bo, W1, b1, W2, b2):
    raise NotImplementedError("write your pallas kernel here")


# speedup vs baseline: 1.0774x; 1.0774x over previous
import jax
import jax.numpy as jnp
from jax.experimental import pallas as pl

N = 10000
E = 160000
F = 256
FE = 16
H = 4
DH = 64
D = 256


def _head_kernel(gr_ref, wp_ref, bp_ref, sm_ref, wv_ref, bv_ref, wo_ref,
                 bo_ref, w1_ref, b1_ref, w2_ref, b2_ref, out_ref):
    gp = gr_ref[...] @ wp_ref[...] + bp_ref[...]
    att = (gp @ wv_ref[...] + bv_ref[...]) @ wo_ref[...] + bo_ref[...]
    sm = sm_ref[...]
    fused = jnp.concatenate([sm, att], axis=1)
    hdn = jnp.maximum(fused @ w1_ref[...] + b1_ref[...], 0.0)
    out_ref[...] = hdn @ w2_ref[...] + b2_ref[...]


def kernel(x, edge_index, edge_attr, batch, Ws, Wd, We, att, gat_b, Wp, bp,
           smiles_repr, Wq, bq, Wk, bk, Wv, bv, Wo, bo, W1, b1, W2, b2):
    src = edge_index[0]
    dst = edge_index[1]
    hs = x @ Ws
    hd = x @ Wd
    he = edge_attr @ We
    v = hs[src] + he
    m = (v + hd[dst]).reshape(E, H, DH)
    score = jnp.sum(jnp.where(m > 0, m, 0.2 * m) * att[None, :, :], axis=-1)
    ex = jnp.exp(score)
    denom = jax.ops.segment_sum(ex, dst, num_segments=N)
    num = jax.ops.segment_sum(v.reshape(E, H, DH) * ex[:, :, None], dst,
                              num_segments=N)
    out = num / (denom + 1e-16)[:, :, None]
    h = out.reshape(N, H * DH) + gat_b
    h = jnp.where(h > 0, h, jnp.expm1(h))
    graph_repr = jnp.mean(h, axis=0, keepdims=True)

    return pl.pallas_call(
        _head_kernel,
        out_shape=jax.ShapeDtypeStruct((1, 1), jnp.float32),
    )(graph_repr, Wp, bp[None, :], smiles_repr, Wv, bv[None, :], Wo,
      bo[None, :], W1, b1[None, :], W2, b2[None, :])


# speedup vs baseline: 13.7583x; 12.7698x over previous
import jax
import jax.numpy as jnp
from jax import lax
from jax.experimental import pallas as pl
from jax.experimental.pallas import tpu as pltpu
from jax.experimental.pallas import tpu_sc as plsc

N = 10000
NP = 10240
E = 160000
F = 256
FE = 16
H = 4
DH = 64
D = 256

NS = 16
EPT = E // NS
C = 40
NCH = EPT // C
NRT = NP // NS
DRT = NRT // 16
RB = 640
NB = (2 * NP) // RB


def _proj_nodes_kernel(x_ref, ws_ref, wd_ref, hs_ref, hd_ref):
    xb = x_ref[...]
    hs_ref[...] = xb @ ws_ref[...]
    hd_ref[...] = xb @ wd_ref[...]


def _proj_edges_kernel(ea_ref, we_ref, he_ref):
    he_ref[...] = ea_ref[...] @ we_ref[...]


def _node_tables(x, Ws, Wd):
    return pl.pallas_call(
        _proj_nodes_kernel,
        grid=(2, 25),
        in_specs=[
            pl.BlockSpec((400, F), lambda j, i: (i, 0)),
            pl.BlockSpec((F, 128), lambda j, i: (0, j)),
            pl.BlockSpec((F, 128), lambda j, i: (0, j)),
        ],
        out_specs=[
            pl.BlockSpec((400, 128), lambda j, i: (j * 25 + i, 0)),
            pl.BlockSpec((400, 128), lambda j, i: (j * 25 + i, 0)),
        ],
        out_shape=[
            jax.ShapeDtypeStruct((2 * N, 128), jnp.float32),
            jax.ShapeDtypeStruct((2 * N, 128), jnp.float32),
        ],
    )(x, Ws, Wd)


def _edge_table(edge_attr, We):
    return pl.pallas_call(
        _proj_edges_kernel,
        grid=(2, 80),
        in_specs=[
            pl.BlockSpec((2000, FE), lambda j, i: (i, 0)),
            pl.BlockSpec((FE, 128), lambda j, i: (0, j)),
        ],
        out_specs=pl.BlockSpec((2000, 128), lambda j, i: (j * 80 + i, 0)),
        out_shape=jax.ShapeDtypeStruct((2 * E, 128), jnp.float32),
    )(edge_attr, We)


def _sc_edge_kernel(hs_hbm, hd_hbm, he_hbm, src_hbm, dst_hbm, att_hbm,
                    num_out, den_out,
                    srcv, dstv, idxs, idxd, idxq, hsg, hdg, heg, deng,
                    attv, num_acc, den_acc, sem1, sem2):
    c = lax.axis_index("c")
    s = lax.axis_index("s")
    lanes = lax.iota(jnp.int32, 16)

    def zero_body(i, zcarry):
        for r in range(8):
            hsg[i, pl.ds(r * 16, 16)] = jnp.zeros((16,), jnp.float32)
            deng[i, pl.ds(r * 16, 16)] = jnp.zeros((16,), jnp.float32)
        return zcarry

    lax.fori_loop(0, C, zero_body, 0)

    def init_body(j, icarry):
        pltpu.sync_copy(hsg, num_acc.at[pl.ds(s * NRT + j * C, C)])
        return icarry

    lax.fori_loop(0, NRT // C, init_body, 0)
    pltpu.sync_copy(deng, den_acc.at[pl.ds(s * DRT, DRT)])
    pltpu.sync_copy(att_hbm, attv)
    plsc.subcore_barrier()

    att_vecs = [jnp.where(c == 0, attv[0, pl.ds(r * 16, 16)],
                          attv[1, pl.ds(r * 16, 16)]) for r in range(8)]
    cN = c * N

    def chunk_body(k, carry):
        e0 = s * EPT + k * C
        pltpu.sync_copy(src_hbm.at[pl.ds(e0, C)], srcv)
        pltpu.sync_copy(dst_hbm.at[pl.ds(e0, C)], dstv)
        for i in range(C // 16 + 1):
            sl = pl.ds(min(i * 16, C - 16), 16)
            idxs[sl] = srcv[sl] + cN
            idxd[sl] = dstv[sl] + cN
            idxq[sl] = lax.shift_right_logical(dstv[sl], 4)
        cps = pltpu.async_copy(hs_hbm.at[idxs], hsg, sem1)
        cpd = pltpu.async_copy(hd_hbm.at[idxd], hdg, sem2)
        pltpu.sync_copy(he_hbm.at[pl.ds(c * E + e0, C)], heg)
        cps.wait()
        cpd.wait()

        for l in range(C):
            gbase = min((l // 16) * 16, C - 16)
            dstg = dstv[pl.ds(gbase, 16)]
            s0 = jnp.zeros((16,), jnp.float32)
            s1 = jnp.zeros((16,), jnp.float32)
            hvs = []
            for r in range(8):
                sl = pl.ds(r * 16, 16)
                hv = hsg[l, sl] + heg[l, sl]
                hvs.append(hv)
                m = hv + hdg[l, sl]
                lr = jnp.maximum(m, 0.2 * m)
                if r < 4:
                    s0 = s0 + lr * att_vecs[r]
                else:
                    s1 = s1 + lr * att_vecs[r]
            p0 = jnp.exp(jnp.full((16,), jnp.sum(s0), jnp.float32))
            p1 = jnp.exp(jnp.full((16,), jnp.sum(s1), jnp.float32))
            for r in range(8):
                sl = pl.ds(r * 16, 16)
                hsg[l, sl] = hvs[r] * (p0 if r < 4 else p1)
            pos0 = (dstg[l - gbase] & 15) * 2
            pos0v = jnp.full((16,), pos0, jnp.int32)
            v0 = (jnp.where(lanes == pos0v, p0, 0.0)
                  + jnp.where(lanes == pos0v + 1, p1, 0.0))
            v1 = (jnp.where(lanes + 16 == pos0v, p0, 0.0)
                  + jnp.where(lanes + 16 == pos0v + 1, p1, 0.0))
            deng[l, pl.ds(0, 16)] = v0
            deng[l, pl.ds(16, 16)] = v1

        pltpu.sync_copy(hsg, num_acc.at[dstv], add=True)
        pltpu.sync_copy(deng, den_acc.at[idxq], add=True)
        return carry

    lax.fori_loop(0, NCH, chunk_body, 0)
    plsc.subcore_barrier()

    def dump_body(j, dcarry):
        o = s * NRT + j * C
        pltpu.sync_copy(num_acc.at[pl.ds(o, C)], hsg)
        pltpu.sync_copy(hsg, num_out.at[pl.ds(c * NP + o, C)])
        return dcarry

    lax.fori_loop(0, NRT // C, dump_body, 0)
    pltpu.sync_copy(den_acc.at[pl.ds(s * DRT, DRT)], deng)
    pltpu.sync_copy(deng, den_out.at[pl.ds(c * (NP // 16) + s * DRT, DRT)])


def _sc_edge_phase(hs_h, hd_h, he_h, src, dst, att2):
    run = pl.kernel(
        _sc_edge_kernel,
        mesh=plsc.VectorSubcoreMesh(core_axis_name="c", subcore_axis_name="s"),
        compiler_params=pltpu.CompilerParams(needs_layout_passes=False),
        out_type=[
            jax.ShapeDtypeStruct((2 * NP, 128), jnp.float32),
            jax.ShapeDtypeStruct((2 * (NP // 16), 128), jnp.float32),
        ],
        scratch_types=[
            pltpu.VMEM((C,), jnp.int32),
            pltpu.VMEM((C,), jnp.int32),
            pltpu.VMEM((C,), jnp.int32),
            pltpu.VMEM((C,), jnp.int32),
            pltpu.VMEM((C,), jnp.int32),
            pltpu.VMEM((C, 128), jnp.float32),
            pltpu.VMEM((C, 128), jnp.float32),
            pltpu.VMEM((C, 128), jnp.float32),
            pltpu.VMEM((C, 128), jnp.float32),
            pltpu.VMEM((2, 128), jnp.float32),
            pltpu.VMEM_SHARED((NP, 128), jnp.float32),
            pltpu.VMEM_SHARED((NP // 16, 128), jnp.float32),
            pltpu.SemaphoreType.DMA,
            pltpu.SemaphoreType.DMA,
        ],
    )
    return run(hs_h, hd_h, he_h, src, dst, att2)


def _finalize_kernel(num_ref, den_ref, gb_ref, wp_ref, bp_ref, sm_ref,
                     wv_ref, bv_ref, wo_ref, bo_ref, w1_ref, b1_ref,
                     w2_ref, b2_ref, out_ref, acc_ref):
    i = pl.program_id(0)

    @pl.when(i == 0)
    def _():
        acc_ref[...] = jnp.zeros_like(acc_ref)

    num = num_ref[...]
    den = den_ref[...]
    d0 = jnp.broadcast_to(den[:, 0:1], (RB, 64))
    d1 = jnp.broadcast_to(den[:, 1:2], (RB, 64))
    denr = jnp.concatenate([d0, d1], axis=1) + 1e-16
    half = i // (NB // 2)
    gb = gb_ref[...]
    gbrow = jnp.where(half == 0, gb[0:1, :], gb[1:2, :])
    hval = num / denr + gbrow
    hval = jnp.where(hval > 0, hval, jnp.exp(jnp.minimum(hval, 0.0)) - 1.0)
    rowid = jax.lax.broadcasted_iota(jnp.int32, (RB, 128), 0)
    node = (i % (NB // 2)) * RB + rowid
    hval = jnp.where(node < N, hval, 0.0)
    colsum = jnp.sum(hval, axis=0, keepdims=True)
    acc_ref[pl.ds(half, 1), :] += colsum

    @pl.when(i == NB - 1)
    def _():
        gr = acc_ref[...] / jnp.float32(N)
        gp = (gr[0:1, :] @ wp_ref[0:128, :]
              + gr[1:2, :] @ wp_ref[128:256, :] + bp_ref[...])
        att = (gp @ wv_ref[...] + bv_ref[...]) @ wo_ref[...] + bo_ref[...]
        fused = jnp.concatenate([sm_ref[...], att], axis=1)
        hdn = jnp.maximum(fused @ w1_ref[...] + b1_ref[...], 0.0)
        out_ref[...] = hdn @ w2_ref[...] + b2_ref[...]


def _finalize(num_h, den_h, gb2, Wp, bp, sm, Wv, bv, Wo, bo, W1, b1, W2, b2):
    full = lambda r, c: pl.BlockSpec((r, c), lambda i: (0, 0))
    return pl.pallas_call(
        _finalize_kernel,
        grid=(NB,),
        in_specs=[
            pl.BlockSpec((RB, 128), lambda i: (i, 0)),
            pl.BlockSpec((RB, 16), lambda i: (i, 0)),
            pl.BlockSpec((2, 128), lambda i: (0, 0)),
            full(D, D), full(1, D), full(1, D),
            full(D, D), full(1, D), full(D, D), full(1, D),
            full(2 * D, D), full(1, D), full(D, 1), full(1, 1),
        ],
        out_specs=pl.BlockSpec((1, 1), lambda i: (0, 0)),
        out_shape=jax.ShapeDtypeStruct((1, 1), jnp.float32),
        scratch_shapes=[pltpu.VMEM((2, 128), jnp.float32)],
    )(num_h, den_h, gb2, Wp, bp, sm, Wv, bv, Wo, bo, W1, b1, W2, b2)


def kernel(x, edge_index, edge_attr, batch, Ws, Wd, We, att, gat_b, Wp, bp,
           smiles_repr, Wq, bq, Wk, bk, Wv, bv, Wo, bo, W1, b1, W2, b2):
    src = edge_index[0]
    dst = edge_index[1]
    att2 = att.reshape(2, 128)
    gb2 = gat_b.reshape(2, 128)
    hs_h, hd_h = _node_tables(x, Ws, Wd)
    he_h = _edge_table(edge_attr, We)
    num_h, den_q = _sc_edge_phase(hs_h, hd_h, he_h, src, dst, att2)
    den4 = den_q.reshape(2 * (NP // 16), 64, 2)[:, 0:16, :]
    den_h = jnp.pad(den4.reshape(2 * NP, 2), ((0, 0), (0, 14)))
    return _finalize(num_h, den_h, gb2, Wp, bp[None, :], smiles_repr,
                     Wv, bv[None, :], Wo, bo[None, :],
                     W1, b1[None, :], W2, b2[None, :])


# speedup vs baseline: 14.9394x; 1.0858x over previous
import jax
import jax.numpy as jnp
from jax import lax
from jax.experimental import pallas as pl
from jax.experimental.pallas import tpu as pltpu
from jax.experimental.pallas import tpu_sc as plsc

N = 10000
NP = 10240
E = 160000
F = 256
FE = 16
H = 4
DH = 64
D = 256

NS = 16
EPT = E // NS
C = 40
NCH = EPT // C
NRT = NP // NS
DRT = NRT // 16
RB = 640
NB = (2 * NP) // RB


def _proj_nodes_kernel(x_ref, ws_ref, wd_ref, hs_ref, hd_ref):
    xb = x_ref[...]
    hs_ref[...] = xb @ ws_ref[...]
    hd_ref[...] = xb @ wd_ref[...]


def _proj_edges_kernel(ea_ref, we_ref, he_ref):
    he_ref[...] = ea_ref[...] @ we_ref[...]


def _node_tables(x, Ws, Wd):
    return pl.pallas_call(
        _proj_nodes_kernel,
        grid=(2, 25),
        in_specs=[
            pl.BlockSpec((400, F), lambda j, i: (i, 0)),
            pl.BlockSpec((F, 128), lambda j, i: (0, j)),
            pl.BlockSpec((F, 128), lambda j, i: (0, j)),
        ],
        out_specs=[
            pl.BlockSpec((400, 128), lambda j, i: (j * 25 + i, 0)),
            pl.BlockSpec((400, 128), lambda j, i: (j * 25 + i, 0)),
        ],
        out_shape=[
            jax.ShapeDtypeStruct((2 * N, 128), jnp.float32),
            jax.ShapeDtypeStruct((2 * N, 128), jnp.float32),
        ],
    )(x, Ws, Wd)


def _edge_table(edge_attr, We):
    return pl.pallas_call(
        _proj_edges_kernel,
        grid=(2, 80),
        in_specs=[
            pl.BlockSpec((2000, FE), lambda j, i: (i, 0)),
            pl.BlockSpec((FE, 128), lambda j, i: (0, j)),
        ],
        out_specs=pl.BlockSpec((2000, 128), lambda j, i: (j * 80 + i, 0)),
        out_shape=jax.ShapeDtypeStruct((2 * E, 128), jnp.float32),
    )(edge_attr, We)


def _sc_edge_kernel(hs_hbm, hd_hbm, he_hbm, src_hbm, dst_hbm, att_hbm,
                    num_out, den_out,
                    srcv, dstv, idxs, idxd, idxq, hsg, hdg, heg, deng,
                    attv, num_acc, den_acc, sem1, sem2, sem3, sem4):
    c = lax.axis_index("c")
    s = lax.axis_index("s")
    lanes = lax.iota(jnp.int32, 16)

    def zero_body(i, zcarry):
        for r in range(8):
            hsg[i, pl.ds(r * 16, 16)] = jnp.zeros((16,), jnp.float32)
            deng[i, pl.ds(r * 16, 16)] = jnp.zeros((16,), jnp.float32)
        return zcarry

    lax.fori_loop(0, C, zero_body, 0)

    def init_body(j, icarry):
        pltpu.sync_copy(hsg, num_acc.at[pl.ds(s * NRT + j * C, C)])
        return icarry

    lax.fori_loop(0, NRT // C, init_body, 0)
    pltpu.sync_copy(deng, den_acc.at[pl.ds(s * DRT, DRT)])
    pltpu.sync_copy(att_hbm, attv)
    plsc.subcore_barrier()

    att_vecs = [jnp.where(c == 0, attv[0, pl.ds(r * 16, 16)],
                          attv[1, pl.ds(r * 16, 16)]) for r in range(8)]
    cN = c * N

    def chunk_body(k, carry):
        e0 = s * EPT + k * C
        pltpu.sync_copy(src_hbm.at[pl.ds(e0, C)], srcv)
        pltpu.sync_copy(dst_hbm.at[pl.ds(e0, C)], dstv)
        for i in range(C // 16 + 1):
            sl = pl.ds(min(i * 16, C - 16), 16)
            idxs[sl] = srcv[sl] + cN
            idxd[sl] = dstv[sl] + cN
            idxq[sl] = lax.shift_right_logical(dstv[sl], 4)

        @pl.when(k > 0)
        def _():
            pltpu.make_async_copy(num_acc.at[pl.ds(0, C)], hsg, sem3).wait()
            pltpu.make_async_copy(den_acc.at[pl.ds(0, C)], deng, sem4).wait()

        cps = pltpu.async_copy(hs_hbm.at[idxs], hsg, sem1)
        cpd = pltpu.async_copy(hd_hbm.at[idxd], hdg, sem2)
        pltpu.sync_copy(he_hbm.at[pl.ds(c * E + e0, C)], heg)
        cps.wait()
        cpd.wait()

        for l in range(C):
            gbase = min((l // 16) * 16, C - 16)
            dstg = dstv[pl.ds(gbase, 16)]
            s0 = jnp.zeros((16,), jnp.float32)
            s1 = jnp.zeros((16,), jnp.float32)
            hvs = []
            for r in range(8):
                sl = pl.ds(r * 16, 16)
                hv = hsg[l, sl] + heg[l, sl]
                hvs.append(hv)
                m = hv + hdg[l, sl]
                lr = jnp.maximum(m, 0.2 * m)
                if r < 4:
                    s0 = s0 + lr * att_vecs[r]
                else:
                    s1 = s1 + lr * att_vecs[r]
            p0 = jnp.exp(jnp.full((16,), jnp.sum(s0), jnp.float32))
            p1 = jnp.exp(jnp.full((16,), jnp.sum(s1), jnp.float32))
            for r in range(8):
                sl = pl.ds(r * 16, 16)
                hsg[l, sl] = hvs[r] * (p0 if r < 4 else p1)
            pos0 = (dstg[l - gbase] & 15) * 2
            pos0v = jnp.full((16,), pos0, jnp.int32)
            v0 = (jnp.where(lanes == pos0v, p0, 0.0)
                  + jnp.where(lanes == pos0v + 1, p1, 0.0))
            v1 = (jnp.where(lanes + 16 == pos0v, p0, 0.0)
                  + jnp.where(lanes + 16 == pos0v + 1, p1, 0.0))
            deng[l, pl.ds(0, 16)] = v0
            deng[l, pl.ds(16, 16)] = v1

        pltpu.async_copy(hsg, num_acc.at[dstv], sem3, add=True)
        pltpu.async_copy(deng, den_acc.at[idxq], sem4, add=True)
        return carry

    lax.fori_loop(0, NCH, chunk_body, 0)
    pltpu.make_async_copy(num_acc.at[pl.ds(0, C)], hsg, sem3).wait()
    pltpu.make_async_copy(den_acc.at[pl.ds(0, C)], deng, sem4).wait()
    plsc.subcore_barrier()

    def dump_body(j, dcarry):
        o = s * NRT + j * C
        pltpu.sync_copy(num_acc.at[pl.ds(o, C)], hsg)
        pltpu.sync_copy(hsg, num_out.at[pl.ds(c * NP + o, C)])
        return dcarry

    lax.fori_loop(0, NRT // C, dump_body, 0)
    pltpu.sync_copy(den_acc.at[pl.ds(s * DRT, DRT)], deng)
    pltpu.sync_copy(deng, den_out.at[pl.ds(c * (NP // 16) + s * DRT, DRT)])


def _sc_edge_phase(hs_h, hd_h, he_h, src, dst, att2):
    run = pl.kernel(
        _sc_edge_kernel,
        mesh=plsc.VectorSubcoreMesh(core_axis_name="c", subcore_axis_name="s"),
        compiler_params=pltpu.CompilerParams(needs_layout_passes=False),
        out_type=[
            jax.ShapeDtypeStruct((2 * NP, 128), jnp.float32),
            jax.ShapeDtypeStruct((2 * (NP // 16), 128), jnp.float32),
        ],
        scratch_types=[
            pltpu.VMEM((C,), jnp.int32),
            pltpu.VMEM((C,), jnp.int32),
            pltpu.VMEM((C,), jnp.int32),
            pltpu.VMEM((C,), jnp.int32),
            pltpu.VMEM((C,), jnp.int32),
            pltpu.VMEM((C, 128), jnp.float32),
            pltpu.VMEM((C, 128), jnp.float32),
            pltpu.VMEM((C, 128), jnp.float32),
            pltpu.VMEM((C, 128), jnp.float32),
            pltpu.VMEM((2, 128), jnp.float32),
            pltpu.VMEM_SHARED((NP, 128), jnp.float32),
            pltpu.VMEM_SHARED((NP // 16, 128), jnp.float32),
            pltpu.SemaphoreType.DMA,
            pltpu.SemaphoreType.DMA,
            pltpu.SemaphoreType.DMA,
            pltpu.SemaphoreType.DMA,
        ],
    )
    return run(hs_h, hd_h, he_h, src, dst, att2)


def _finalize_kernel(num_ref, den_ref, gb_ref, wp_ref, bp_ref, sm_ref,
                     wv_ref, bv_ref, wo_ref, bo_ref, w1_ref, b1_ref,
                     w2_ref, b2_ref, out_ref, acc_ref):
    i = pl.program_id(0)

    @pl.when(i == 0)
    def _():
        acc_ref[...] = jnp.zeros_like(acc_ref)

    num = num_ref[...]
    den = den_ref[...]
    d0 = jnp.broadcast_to(den[:, 0:1], (RB, 64))
    d1 = jnp.broadcast_to(den[:, 1:2], (RB, 64))
    denr = jnp.concatenate([d0, d1], axis=1) + 1e-16
    half = i // (NB // 2)
    gb = gb_ref[...]
    gbrow = jnp.where(half == 0, gb[0:1, :], gb[1:2, :])
    hval = num / denr + gbrow
    hval = jnp.where(hval > 0, hval, jnp.exp(jnp.minimum(hval, 0.0)) - 1.0)
    rowid = jax.lax.broadcasted_iota(jnp.int32, (RB, 128), 0)
    node = (i % (NB // 2)) * RB + rowid
    hval = jnp.where(node < N, hval, 0.0)
    colsum = jnp.sum(hval, axis=0, keepdims=True)
    acc_ref[pl.ds(half, 1), :] += colsum

    @pl.when(i == NB - 1)
    def _():
        gr = acc_ref[...] / jnp.float32(N)
        gp = (gr[0:1, :] @ wp_ref[0:128, :]
              + gr[1:2, :] @ wp_ref[128:256, :] + bp_ref[...])
        att = (gp @ wv_ref[...] + bv_ref[...]) @ wo_ref[...] + bo_ref[...]
        fused = jnp.concatenate([sm_ref[...], att], axis=1)
        hdn = jnp.maximum(fused @ w1_ref[...] + b1_ref[...], 0.0)
        out_ref[...] = hdn @ w2_ref[...] + b2_ref[...]


def _finalize(num_h, den_h, gb2, Wp, bp, sm, Wv, bv, Wo, bo, W1, b1, W2, b2):
    full = lambda r, c: pl.BlockSpec((r, c), lambda i: (0, 0))
    return pl.pallas_call(
        _finalize_kernel,
        grid=(NB,),
        in_specs=[
            pl.BlockSpec((RB, 128), lambda i: (i, 0)),
            pl.BlockSpec((RB, 16), lambda i: (i, 0)),
            pl.BlockSpec((2, 128), lambda i: (0, 0)),
            full(D, D), full(1, D), full(1, D),
            full(D, D), full(1, D), full(D, D), full(1, D),
            full(2 * D, D), full(1, D), full(D, 1), full(1, 1),
        ],
        out_specs=pl.BlockSpec((1, 1), lambda i: (0, 0)),
        out_shape=jax.ShapeDtypeStruct((1, 1), jnp.float32),
        scratch_shapes=[pltpu.VMEM((2, 128), jnp.float32)],
    )(num_h, den_h, gb2, Wp, bp, sm, Wv, bv, Wo, bo, W1, b1, W2, b2)


def kernel(x, edge_index, edge_attr, batch, Ws, Wd, We, att, gat_b, Wp, bp,
           smiles_repr, Wq, bq, Wk, bk, Wv, bv, Wo, bo, W1, b1, W2, b2):
    src = edge_index[0]
    dst = edge_index[1]
    att2 = att.reshape(2, 128)
    gb2 = gat_b.reshape(2, 128)
    hs_h, hd_h = _node_tables(x, Ws, Wd)
    he_h = _edge_table(edge_attr, We)
    num_h, den_q = _sc_edge_phase(hs_h, hd_h, he_h, src, dst, att2)
    den4 = den_q.reshape(2 * (NP // 16), 64, 2)[:, 0:16, :]
    den_h = jnp.pad(den4.reshape(2 * NP, 2), ((0, 0), (0, 14)))
    return _finalize(num_h, den_h, gb2, Wp, bp[None, :], smiles_repr,
                     Wv, bv[None, :], Wo, bo[None, :],
                     W1, b1[None, :], W2, b2[None, :])
